# Initial kernel scaffold; baseline (speedup 1.0000x reference)
import jax
import jax.numpy as jnp
from jax.experimental import pallas as pl


def kernel(z_i, z_j, z_weak, labels, anchors):
    raise NotImplementedError("write your pallas kernel here")


# speedup vs baseline: 2.3389x; 2.3389x over previous
import jax
import jax.numpy as jnp
from jax.experimental import pallas as pl
from jax.experimental.pallas import tpu as pltpu

_B = 16384
_D = 1024
_C = 1000
_BB = 512


def _tc_body(lab_ref, zi_ref, zj_ref, anc_ref, out_ref):
    g = pl.program_id(0)
    zi = zi_ref[...]
    zj = zj_ref[...]
    lab = lab_ref[0, 0, :]
    ns_i = jnp.sum(zi * zi, axis=1, keepdims=True)
    ns_j = jnp.sum(zj * zj, axis=1, keepdims=True)
    rs_i = 1.0 / jnp.maximum(jnp.sqrt(ns_i), 1e-12)
    rs_j = 1.0 / jnp.maximum(jnp.sqrt(ns_j), 1e-12)
    w = zi * rs_i + zj * rs_j
    onehot = (lab[:, None] == jax.lax.broadcasted_iota(jnp.int32, (_BB, _C), 1))
    onehot = onehot.astype(jnp.bfloat16)
    gathered = jnp.dot(onehot, anc_ref[...], preferred_element_type=jnp.float32)
    blk = jnp.sum(w * gathered).reshape(1, 1)

    @pl.when(g == 0)
    def _():
        out_ref[...] = jnp.zeros_like(out_ref)

    out_ref[...] += blk


def kernel(z_i, z_j, z_weak, labels, anchors):
    lab3 = labels.astype(jnp.int32).reshape(_B // _BB, 1, _BB)
    anc_bf = anchors.astype(jnp.bfloat16)
    acc = pl.pallas_call(
        _tc_body,
        grid=(_B // _BB,),
        in_specs=[
            pl.BlockSpec((1, 1, _BB), lambda g: (g, 0, 0)),
            pl.BlockSpec((_BB, _D), lambda g: (g, 0)),
            pl.BlockSpec((_BB, _D), lambda g: (g, 0)),
            pl.BlockSpec((_C, _D), lambda g: (0, 0)),
        ],
        out_specs=pl.BlockSpec((1, 1), lambda g: (0, 0)),
        out_shape=jax.ShapeDtypeStruct((1, 1), jnp.float32),
    )(lab3, z_i, z_j, anc_bf)
    return 1.0 - acc[0, 0] / (2.0 * _B)
